# Initial kernel scaffold; baseline (speedup 1.0000x reference)
import jax
import jax.numpy as jnp
from jax.experimental import pallas as pl


def kernel(x, edge_index, W1, b1, W2, b2, W3, b3):
    raise NotImplementedError("write your pallas kernel here")


# speedup vs baseline: 10.5184x; 10.5184x over previous
import functools

import jax
import jax.numpy as jnp
from jax import lax
from jax.experimental import pallas as pl
from jax.experimental.pallas import tpu as pltpu
from jax.experimental.pallas import tpu_sc as plsc

N = 10000
NP = 10240
E = 160000
NC = 2
NS = 16
NW = NC * NS
EPW = 5120
CH = 128
NCHUNK = EPW // CH
RPT = NP // NS
ZCP = RPT // CH


def _sc_mesh():
    return plsc.VectorSubcoreMesh(
        core_axis_name="c", subcore_axis_name="s", num_cores=NC, num_subcores=NS
    )


def _make_deg():
    def body(dst_hbm, ones_hbm, zeros_hbm, degp_hbm, idx_v, ones_v, zb_v, acc):
        c = lax.axis_index("c")
        s = lax.axis_index("s")
        wid = c * NS + s
        pltpu.sync_copy(dst_hbm.at[wid], idx_v)
        pltpu.sync_copy(ones_hbm, ones_v)
        pltpu.sync_copy(zeros_hbm, zb_v)

        row0 = s * RPT
        for k in range(ZCP):
            pltpu.sync_copy(zb_v, acc.at[pl.ds(row0 + k * CH, CH)])
        plsc.subcore_barrier()

        def _scatter(j, _):
            pltpu.sync_copy(ones_v, acc.at[idx_v.at[j]], add=True)
            return _

        lax.fori_loop(0, NCHUNK, _scatter, None)
        plsc.subcore_barrier()

        for k in range(ZCP):
            pltpu.sync_copy(acc.at[pl.ds(row0 + k * CH, CH)], zb_v)
            pltpu.sync_copy(zb_v, degp_hbm.at[c, pl.ds(row0 + k * CH, CH)])

    return pl.kernel(
        body,
        out_type=jax.ShapeDtypeStruct((NC, NP, 16), jnp.float32),
        mesh=_sc_mesh(),
        compiler_params=pltpu.CompilerParams(use_tc_tiling_on_sc=False),
        scratch_types=[
            pltpu.VMEM((NCHUNK, CH), jnp.int32),
            pltpu.VMEM((CH, 16), jnp.float32),
            pltpu.VMEM((CH, 16), jnp.float32),
            pltpu.VMEM_SHARED((NP, 16), jnp.float32),
        ],
    )


def _make_spmm(D):
    def body(u_hbm, src_hbm, dst_hbm, zeros_hbm, out_hbm,
             sidx, didx, buf0, buf1, zb, acc, sem0, sem1):
        c = lax.axis_index("c")
        s = lax.axis_index("s")
        wid = c * NS + s
        pltpu.sync_copy(src_hbm.at[wid], sidx)
        pltpu.sync_copy(dst_hbm.at[wid], didx)
        pltpu.sync_copy(zeros_hbm, zb)

        row0 = s * RPT
        for k in range(ZCP):
            pltpu.sync_copy(zb, acc.at[pl.ds(row0 + k * CH, CH)])
        plsc.subcore_barrier()

        def step(j, _):
            j0 = 2 * j
            j1 = 2 * j + 1
            d0 = pltpu.async_copy(u_hbm.at[sidx.at[j0]], buf0, sem0)
            d1 = pltpu.async_copy(u_hbm.at[sidx.at[j1]], buf1, sem1)
            d0.wait()
            pltpu.sync_copy(buf0, acc.at[didx.at[j0]], add=True)
            d1.wait()
            pltpu.sync_copy(buf1, acc.at[didx.at[j1]], add=True)
            return _

        lax.fori_loop(0, NCHUNK // 2, step, None)
        plsc.subcore_barrier()

        for k in range(ZCP):
            pltpu.sync_copy(acc.at[pl.ds(row0 + k * CH, CH)], buf0)
            pltpu.sync_copy(buf0, out_hbm.at[c, pl.ds(row0 + k * CH, CH)])

    return pl.kernel(
        body,
        out_type=jax.ShapeDtypeStruct((NC, NP, D), jnp.float32),
        mesh=_sc_mesh(),
        compiler_params=pltpu.CompilerParams(use_tc_tiling_on_sc=False),
        scratch_types=[
            pltpu.VMEM((NCHUNK, CH), jnp.int32),
            pltpu.VMEM((NCHUNK, CH), jnp.int32),
            pltpu.VMEM((CH, D), jnp.float32),
            pltpu.VMEM((CH, D), jnp.float32),
            pltpu.VMEM((CH, D), jnp.float32),
            pltpu.VMEM_SHARED((NP, D), jnp.float32),
            pltpu.SemaphoreType.DMA,
            pltpu.SemaphoreType.DMA,
        ],
    )


MB = 1280
GRID = NP // MB


def _tc1_body(degp, x, w1, u1, dv16):
    deg = degp[0] + degp[1] + 1.0
    dinv = lax.rsqrt(deg)
    dv16[...] = dinv
    d1 = dinv[:, 0:1]
    u1[...] = jnp.dot(x[...], w1[...], preferred_element_type=jnp.float32) * d1


def _tc3_body(dv16, sp, u, w, b, outa, outb):
    i = pl.program_id(0)
    d1 = dv16[:, 0:1]
    t = d1 * (sp[0] + sp[1] + u[...])
    h = jnp.dot(t, w[...], preferred_element_type=jnp.float32) + b[...]
    h = jnp.maximum(h, 0.0)
    rows = i * MB + lax.broadcasted_iota(jnp.int32, (MB, 1), 0)
    u3 = jnp.where(rows < N, d1 * h, 0.0)
    outa[...] = u3[:, :64]
    outb[...] = u3[:, 64:]


def _tc2_body(dv16, sp, u, b, out):
    i = pl.program_id(0)
    d1 = dv16[:, 0:1]
    t = d1 * (sp[0] + sp[1] + u[...]) + b[...]
    h = jnp.maximum(t, 0.0)
    rows = i * MB + lax.broadcasted_iota(jnp.int32, (MB, 1), 0)
    out[...] = jnp.where(rows < N, d1 * h, 0.0)


def _tc4_body(dv16, spa, spb, ua, ub, wa, wb, b, out):
    d1 = dv16[:, 0:1]
    ta = d1 * (spa[0] + spa[1] + ua[...])
    tb = d1 * (spb[0] + spb[1] + ub[...])
    out[...] = (jnp.dot(ta, wa[...], preferred_element_type=jnp.float32)
                + jnp.dot(tb, wb[...], preferred_element_type=jnp.float32)
                + b[...])


def _row_spec(D):
    return pl.BlockSpec((MB, D), lambda i: (i, 0))


def _pair_spec(D):
    return pl.BlockSpec((NC, MB, D), lambda i: (0, i, 0))


def _full_spec(shape):
    return pl.BlockSpec(shape, lambda i: tuple(0 for _ in shape))


def kernel(x, edge_index, W1, b1, W2, b2, W3, b3):
    f32 = jnp.float32
    src = edge_index[0]
    dst = edge_index[1]
    pad = jnp.full((NW * EPW - E,), N, dtype=jnp.int32)
    src3 = jnp.concatenate([src, pad]).reshape(NW, NCHUNK, CH)
    dst3 = jnp.concatenate([dst, pad]).reshape(NW, NCHUNK, CH)
    x_pad = jnp.zeros((NP, 128), f32).at[:N].set(x)

    ones16 = jnp.ones((CH, 16), f32)
    zeros16 = jnp.zeros((CH, 16), f32)
    zeros64 = jnp.zeros((CH, 64), f32)

    degp = _make_deg()(dst3, ones16, zeros16)

    u1, dv16 = pl.pallas_call(
        _tc1_body,
        grid=(GRID,),
        in_specs=[_pair_spec(16), _row_spec(128), _full_spec((128, 64))],
        out_specs=[_row_spec(64), _row_spec(16)],
        out_shape=[
            jax.ShapeDtypeStruct((NP, 64), f32),
            jax.ShapeDtypeStruct((NP, 16), f32),
        ],
    )(degp, x_pad, W1)

    spmm64 = _make_spmm(64)

    s1 = spmm64(u1, src3, dst3, zeros64)
    u2 = pl.pallas_call(
        _tc2_body,
        grid=(GRID,),
        in_specs=[_row_spec(16), _pair_spec(64), _row_spec(64),
                  _full_spec((1, 64))],
        out_specs=_row_spec(64),
        out_shape=jax.ShapeDtypeStruct((NP, 64), f32),
    )(dv16, s1, u1, b1.reshape(1, 64))

    s2 = spmm64(u2, src3, dst3, zeros64)
    u3a, u3b = pl.pallas_call(
        _tc3_body,
        grid=(GRID,),
        in_specs=[_row_spec(16), _pair_spec(64), _row_spec(64),
                  _full_spec((64, 128)), _full_spec((1, 128))],
        out_specs=[_row_spec(64), _row_spec(64)],
        out_shape=[jax.ShapeDtypeStruct((NP, 64), f32),
                   jax.ShapeDtypeStruct((NP, 64), f32)],
    )(dv16, s2, u2, W2, b2.reshape(1, 128))

    s3a = spmm64(u3a, src3, dst3, zeros64)
    s3b = spmm64(u3b, src3, dst3, zeros64)
    out = pl.pallas_call(
        _tc4_body,
        grid=(GRID,),
        in_specs=[_row_spec(16), _pair_spec(64), _pair_spec(64),
                  _row_spec(64), _row_spec(64),
                  _full_spec((64, 1024)), _full_spec((64, 1024)),
                  _full_spec((1, 1024))],
        out_specs=_row_spec(1024),
        out_shape=jax.ShapeDtypeStruct((NP, 1024), f32),
    )(dv16, s3a, s3b, u3a, u3b, W3[:64], W3[64:], b3.reshape(1, 1024))

    return out[:N].reshape(N, 32, 32)


# speedup vs baseline: 11.7981x; 1.1217x over previous
import functools

import jax
import jax.numpy as jnp
from jax import lax
from jax.experimental import pallas as pl
from jax.experimental.pallas import tpu as pltpu
from jax.experimental.pallas import tpu_sc as plsc

N = 10000
NP = 10240
E = 160000
NC = 2
NS = 16
NW = NC * NS
EPW = 5120
CH = 128
NCHUNK = EPW // CH
RPT = NP // NS
ZCP = RPT // CH


def _sc_mesh():
    return plsc.VectorSubcoreMesh(
        core_axis_name="c", subcore_axis_name="s", num_cores=NC, num_subcores=NS
    )


def _make_deg():
    def body(dst_hbm, ones_hbm, zeros_hbm, degp_hbm, idx_v, ones_v, zb_v, acc):
        c = lax.axis_index("c")
        s = lax.axis_index("s")
        wid = c * NS + s
        pltpu.sync_copy(dst_hbm.at[wid], idx_v)
        pltpu.sync_copy(ones_hbm, ones_v)
        pltpu.sync_copy(zeros_hbm, zb_v)

        row0 = s * RPT
        for k in range(ZCP):
            pltpu.sync_copy(zb_v, acc.at[pl.ds(row0 + k * CH, CH)])
        plsc.subcore_barrier()

        def _scatter(j, _):
            pltpu.sync_copy(ones_v, acc.at[idx_v.at[j]], add=True)
            return _

        lax.fori_loop(0, NCHUNK, _scatter, None)
        plsc.subcore_barrier()

        for k in range(ZCP):
            pltpu.sync_copy(acc.at[pl.ds(row0 + k * CH, CH)], zb_v)
            pltpu.sync_copy(zb_v, degp_hbm.at[c, pl.ds(row0 + k * CH, CH)])

    return pl.kernel(
        body,
        out_type=jax.ShapeDtypeStruct((NC, NP, 16), jnp.float32),
        mesh=_sc_mesh(),
        compiler_params=pltpu.CompilerParams(use_tc_tiling_on_sc=False),
        scratch_types=[
            pltpu.VMEM((NCHUNK, CH), jnp.int32),
            pltpu.VMEM((CH, 16), jnp.float32),
            pltpu.VMEM((CH, 16), jnp.float32),
            pltpu.VMEM_SHARED((NP, 16), jnp.float32),
        ],
    )


def _make_spmm(D):
    NB = 4

    def body(u_hbm, src_hbm, dst_hbm, zeros_hbm, out_hbm,
             sidx, didx, bufs, zb, acc, sems):
        c = lax.axis_index("c")
        s = lax.axis_index("s")
        wid = c * NS + s
        pltpu.sync_copy(src_hbm.at[wid], sidx)
        pltpu.sync_copy(dst_hbm.at[wid], didx)
        pltpu.sync_copy(zeros_hbm, zb)

        row0 = s * RPT
        for k in range(ZCP):
            pltpu.sync_copy(zb, acc.at[pl.ds(row0 + k * CH, CH)])
        plsc.subcore_barrier()

        for b in range(NB):
            pltpu.async_copy(u_hbm.at[sidx.at[b]], bufs.at[b], sems[b])

        def step(j, _):
            for b in range(NB):
                cj = NB * j + b
                pltpu.make_async_copy(
                    u_hbm.at[sidx.at[cj]], bufs.at[b], sems[b]).wait()
                pltpu.sync_copy(bufs.at[b], acc.at[didx.at[cj]], add=True)
                pltpu.async_copy(
                    u_hbm.at[sidx.at[cj + NB]], bufs.at[b], sems[b])
            return _

        lax.fori_loop(0, NCHUNK // NB - 1, step, None)
        for b in range(NB):
            cj = NCHUNK - NB + b
            pltpu.make_async_copy(
                u_hbm.at[sidx.at[cj]], bufs.at[b], sems[b]).wait()
            pltpu.sync_copy(bufs.at[b], acc.at[didx.at[cj]], add=True)
        plsc.subcore_barrier()

        for k in range(ZCP):
            pltpu.sync_copy(acc.at[pl.ds(row0 + k * CH, CH)], zb)
            pltpu.sync_copy(zb, out_hbm.at[c, pl.ds(row0 + k * CH, CH)])

    return pl.kernel(
        body,
        out_type=jax.ShapeDtypeStruct((NC, NP, D), jnp.float32),
        mesh=_sc_mesh(),
        compiler_params=pltpu.CompilerParams(use_tc_tiling_on_sc=False),
        scratch_types=[
            pltpu.VMEM((NCHUNK, CH), jnp.int32),
            pltpu.VMEM((NCHUNK, CH), jnp.int32),
            pltpu.VMEM((NB, CH, D), jnp.float32),
            pltpu.VMEM((CH, D), jnp.float32),
            pltpu.VMEM_SHARED((NP, D), jnp.float32),
            [pltpu.SemaphoreType.DMA] * NB,
        ],
    )


MB = 1280
GRID = NP // MB


def _tc1_body(degp, x, w1, u1, dv16):
    deg = degp[0] + degp[1] + 1.0
    dinv = lax.rsqrt(deg)
    dv16[...] = dinv
    d1 = dinv[:, 0:1]
    u1[...] = jnp.dot(x[...], w1[...], preferred_element_type=jnp.float32) * d1


def _tc3_body(dv16, sp, u, w, b, outa, outb):
    i = pl.program_id(0)
    d1 = dv16[:, 0:1]
    t = d1 * (sp[0] + sp[1] + u[...])
    h = jnp.dot(t, w[...], preferred_element_type=jnp.float32) + b[...]
    h = jnp.maximum(h, 0.0)
    rows = i * MB + lax.broadcasted_iota(jnp.int32, (MB, 1), 0)
    u3 = jnp.where(rows < N, d1 * h, 0.0)
    outa[...] = u3[:, :64]
    outb[...] = u3[:, 64:]


def _tc2_body(dv16, sp, u, b, out):
    i = pl.program_id(0)
    d1 = dv16[:, 0:1]
    t = d1 * (sp[0] + sp[1] + u[...]) + b[...]
    h = jnp.maximum(t, 0.0)
    rows = i * MB + lax.broadcasted_iota(jnp.int32, (MB, 1), 0)
    out[...] = jnp.where(rows < N, d1 * h, 0.0)


def _tc4_body(dv16, spa, spb, ua, ub, wa, wb, b, out):
    d1 = dv16[:, 0:1]
    ta = d1 * (spa[0] + spa[1] + ua[...])
    tb = d1 * (spb[0] + spb[1] + ub[...])
    out[...] = (jnp.dot(ta, wa[...], preferred_element_type=jnp.float32)
                + jnp.dot(tb, wb[...], preferred_element_type=jnp.float32)
                + b[...])


def _row_spec(D):
    return pl.BlockSpec((MB, D), lambda i: (i, 0))


def _pair_spec(D):
    return pl.BlockSpec((NC, MB, D), lambda i: (0, i, 0))


def _full_spec(shape):
    return pl.BlockSpec(shape, lambda i: tuple(0 for _ in shape))


def kernel(x, edge_index, W1, b1, W2, b2, W3, b3):
    f32 = jnp.float32
    src = edge_index[0]
    dst = edge_index[1]
    pad = jnp.full((NW * EPW - E,), N, dtype=jnp.int32)
    src3 = jnp.concatenate([src, pad]).reshape(NW, NCHUNK, CH)
    dst3 = jnp.concatenate([dst, pad]).reshape(NW, NCHUNK, CH)
    x_pad = jnp.zeros((NP, 128), f32).at[:N].set(x)

    ones16 = jnp.ones((CH, 16), f32)
    zeros16 = jnp.zeros((CH, 16), f32)
    zeros64 = jnp.zeros((CH, 64), f32)

    degp = _make_deg()(dst3, ones16, zeros16)

    u1, dv16 = pl.pallas_call(
        _tc1_body,
        grid=(GRID,),
        in_specs=[_pair_spec(16), _row_spec(128), _full_spec((128, 64))],
        out_specs=[_row_spec(64), _row_spec(16)],
        out_shape=[
            jax.ShapeDtypeStruct((NP, 64), f32),
            jax.ShapeDtypeStruct((NP, 16), f32),
        ],
    )(degp, x_pad, W1)

    spmm64 = _make_spmm(64)

    s1 = spmm64(u1, src3, dst3, zeros64)
    u2 = pl.pallas_call(
        _tc2_body,
        grid=(GRID,),
        in_specs=[_row_spec(16), _pair_spec(64), _row_spec(64),
                  _full_spec((1, 64))],
        out_specs=_row_spec(64),
        out_shape=jax.ShapeDtypeStruct((NP, 64), f32),
    )(dv16, s1, u1, b1.reshape(1, 64))

    s2 = spmm64(u2, src3, dst3, zeros64)
    u3a, u3b = pl.pallas_call(
        _tc3_body,
        grid=(GRID,),
        in_specs=[_row_spec(16), _pair_spec(64), _row_spec(64),
                  _full_spec((64, 128)), _full_spec((1, 128))],
        out_specs=[_row_spec(64), _row_spec(64)],
        out_shape=[jax.ShapeDtypeStruct((NP, 64), f32),
                   jax.ShapeDtypeStruct((NP, 64), f32)],
    )(dv16, s2, u2, W2, b2.reshape(1, 128))

    s3a = spmm64(u3a, src3, dst3, zeros64)
    s3b = spmm64(u3b, src3, dst3, zeros64)
    out = pl.pallas_call(
        _tc4_body,
        grid=(GRID,),
        in_specs=[_row_spec(16), _pair_spec(64), _pair_spec(64),
                  _row_spec(64), _row_spec(64),
                  _full_spec((64, 1024)), _full_spec((64, 1024)),
                  _full_spec((1, 1024))],
        out_specs=_row_spec(1024),
        out_shape=jax.ShapeDtypeStruct((N, 1024), f32),
    )(dv16, s3a, s3b, u3a, u3b, W3[:64], W3[64:], b3.reshape(1, 1024))

    return out.reshape(N, 32, 32)
